# Initial kernel scaffold; baseline (speedup 1.0000x reference)
import jax
import jax.numpy as jnp
from jax.experimental import pallas as pl


def kernel(pred_seg, atom_y, pred_pos_global_node, bindingsite_center, preds_confidence, x_batch, y_batch):
    raise NotImplementedError("write your pallas kernel here")


# speedup vs baseline: 2.9562x; 2.9562x over previous
import functools
import jax
import jax.numpy as jnp
from jax.experimental import pallas as pl
from jax.experimental.pallas import tpu as pltpu

_B = 128
_N_ATOM = 100000
_N_X = 1024
_N_Y = 32768

_YBLK = 512
_GRID = _N_Y // _YBLK
_SEG_PAD = _GRID * 1664
_SEGBLK = _SEG_PAD // _GRID


def _loss_kernel(yt, xt, yb, xb, conf, seg, aty, y0c, out,
                 colmin, acc):
    i = pl.program_id(0)

    @pl.when(i == 0)
    def _init():
        colmin[...] = jnp.full((_N_X,), jnp.inf, jnp.float32)
        acc[0] = 0.0
        acc[1] = 0.0
        acc[2] = 0.0
        acc[3] = 0.0

    d2 = jnp.zeros((_YBLK, _N_X), jnp.float32)
    for c in range(3):
        diff = yt[c, :].reshape(_YBLK, 1) - xt[c, :].reshape(1, _N_X)
        d2 = d2 + diff * diff
    mask = yb[...].reshape(_YBLK, 1) == xb[...].reshape(1, _N_X)
    d2m = jnp.where(mask, d2, jnp.inf)

    ridx = jnp.argmin(d2m, axis=1).astype(jnp.int32)
    onehot = (ridx.reshape(_YBLK, 1) ==
              jax.lax.broadcasted_iota(jnp.int32, (_YBLK, _N_X), 1))
    hub = jnp.zeros((_YBLK,), jnp.float32)
    for c in range(3):
        xg = jnp.sum(jnp.where(onehot, xt[c, :].reshape(1, _N_X), 0.0), axis=1)
        err = yt[c, :] - xg
        a = jnp.abs(err)
        hub = hub + jnp.where(a < 1.0, 0.5 * err * err, a - 0.5)
    acc[0] += jnp.sum(hub)

    colmin[...] = jnp.minimum(colmin[...], jnp.min(d2m, axis=0))

    p = jax.nn.sigmoid(seg[...])
    t = aty[...]
    acc[1] += jnp.sum(p)
    acc[2] += jnp.sum(p * t)
    acc[3] += jnp.sum(t)

    @pl.when(i == _GRID - 1)
    def _fin():
        cm = colmin[...]
        d2y0 = jnp.zeros((_N_X,), jnp.float32)
        for c in range(3):
            dd = xt[c, :] - y0c[c, 0]
            d2y0 = d2y0 + dd * dd
        cm = jnp.where(jnp.isinf(cm), d2y0, cm)
        dist = jnp.sqrt(cm)
        dc = conf[...] - dist
        conf_loss = jnp.sum(dc * dc) / _N_X
        eps = 1e-6
        dice = 1.0 - (2.0 * acc[2] + eps) / (acc[1] + acc[3] + eps)
        hub_mean = acc[0] / (_N_Y * 3.0)
        out[...] = (hub_mean + dice + conf_loss).reshape(1, 1)


@jax.jit
def kernel(pred_seg, atom_y, pred_pos_global_node, bindingsite_center,
           preds_confidence, x_batch, y_batch):
    xt = pred_pos_global_node.T
    yt = bindingsite_center.T
    y0c = yt[:, 0:1]
    seg = jnp.pad(pred_seg[:, 0], (0, _SEG_PAD - _N_ATOM),
                  constant_values=-1e9).reshape(_GRID, 1, _SEGBLK)
    aty = jnp.pad(atom_y, (0, _SEG_PAD - _N_ATOM)).reshape(_GRID, 1, _SEGBLK)
    conf = preds_confidence[:, 0]

    out = pl.pallas_call(
        _loss_kernel,
        grid=(_GRID,),
        in_specs=[
            pl.BlockSpec((3, _YBLK), lambda i: (0, i)),
            pl.BlockSpec((3, _N_X), lambda i: (0, 0)),
            pl.BlockSpec((_YBLK,), lambda i: (i,)),
            pl.BlockSpec((_N_X,), lambda i: (0,)),
            pl.BlockSpec((_N_X,), lambda i: (0,)),
            pl.BlockSpec((1, 1, _SEGBLK), lambda i: (i, 0, 0)),
            pl.BlockSpec((1, 1, _SEGBLK), lambda i: (i, 0, 0)),
            pl.BlockSpec((3, 1), lambda i: (0, 0)),
        ],
        out_specs=pl.BlockSpec((1, 1), lambda i: (0, 0)),
        out_shape=jax.ShapeDtypeStruct((1, 1), jnp.float32),
        scratch_shapes=[
            pltpu.VMEM((_N_X,), jnp.float32),
            pltpu.SMEM((4,), jnp.float32),
        ],
    )(yt, xt, y_batch, x_batch, conf, seg, aty, y0c)
    return out[0, 0]


# speedup vs baseline: 9.2142x; 3.1169x over previous
import jax
import jax.numpy as jnp
from jax import lax
from jax.experimental import pallas as pl
from jax.experimental.pallas import tpu as pltpu
from jax.experimental.pallas import tpu_sc as plsc

_B = 128
_N_ATOM = 100000
_N_X = 1024
_N_Y = 32768
_NW = 32
_YPW = _N_Y // _NW
_NVY = _YPW // 16
_SEGW = 3136
_NVSEG = _SEGW // 16
_SEG_PAD = _NW * _SEGW


def _sc_body(x0_h, x1_h, x2_h, xb_h, y3_h, yb_h, seg_h, aty_h,
             colmin_out, part_out,
             x0_v, x1_v, x2_v, xb_v, xs_v, xe_v,
             yt_v, yb_v, colmin_v, seg_v, aty_v,
             part_v, mrg_v, cm_v, pall_v, pout_v,
             colmin_sh, part_sh):
    cid = lax.axis_index("c")
    sid = lax.axis_index("s")
    wid = cid * 16 + sid

    pltpu.sync_copy(x0_h, x0_v)
    pltpu.sync_copy(x1_h, x1_v)
    pltpu.sync_copy(x2_h, x2_v)
    pltpu.sync_copy(xb_h, xb_v)
    pltpu.sync_copy(y3_h.at[wid], yt_v)
    pltpu.sync_copy(yb_h.at[wid], yb_v)
    pltpu.sync_copy(seg_h.at[wid], seg_v)
    pltpu.sync_copy(aty_h.at[wid], aty_v)

    inf16 = jnp.full((16,), jnp.inf, jnp.float32)
    zero16 = jnp.zeros((16,), jnp.float32)
    lane = lax.iota(jnp.int32, 16)
    lane0 = lane == 0

    def init_body(i, _):
        colmin_v[pl.ds(i * 16, 16)] = inf16
        return 0
    lax.fori_loop(0, _N_X // 16, init_body, 0)

    def bs_body(t, _):
        b = lane + t * 16
        lo = jnp.zeros((16,), jnp.int32)
        hi = jnp.full((16,), _N_X, jnp.int32)
        lo2 = jnp.zeros((16,), jnp.int32)
        hi2 = jnp.full((16,), _N_X, jnp.int32)
        for _unused in range(11):
            mid = jnp.minimum((lo + hi) >> 1, _N_X - 1)
            v = plsc.load_gather(xb_v, [mid])
            p = v < b
            lo = jnp.where(p, mid + 1, lo)
            hi = jnp.where(p, hi, mid)
            mid2 = jnp.minimum((lo2 + hi2) >> 1, _N_X - 1)
            v2 = plsc.load_gather(xb_v, [mid2])
            p2 = v2 <= b
            lo2 = jnp.where(p2, mid2 + 1, lo2)
            hi2 = jnp.where(p2, hi2, mid2)
        xs_v[pl.ds(t * 16, 16)] = lo
        xe_v[pl.ds(t * 16, 16)] = lo2
        return 0
    lax.fori_loop(0, _B // 16, bs_body, 0)

    def yblk(k, hub16):
        yb16 = yb_v[pl.ds(k * 16, 16)]
        y0 = yt_v[pl.ds(k * 16, 16)]
        y1 = yt_v[pl.ds(_YPW + k * 16, 16)]
        y2 = yt_v[pl.ds(2 * _YPW + k * 16, 16)]
        s = plsc.load_gather(xs_v, [yb16])
        e = plsc.load_gather(xe_v, [yb16])
        jlo = jnp.min(s)
        jhi = jnp.max(e)

        def inner(j, carry):
            best, bidx = carry
            j16 = jnp.full((16,), j, jnp.int32)
            a0 = plsc.load_gather(x0_v, [j16])
            a1 = plsc.load_gather(x1_v, [j16])
            a2 = plsc.load_gather(x2_v, [j16])
            d0 = y0 - a0
            d1 = y1 - a1
            d2c = y2 - a2
            dd = d0 * d0 + d1 * d1 + d2c * d2c
            inb = (s <= j) & (j < e)
            ddm = jnp.where(inb, dd, jnp.inf)
            upd = ddm < best
            best = jnp.where(upd, ddm, best)
            bidx = jnp.where(upd, j16, bidx)
            cm = jnp.min(ddm)
            cur = plsc.load_gather(colmin_v, [j16])
            plsc.store_scatter(colmin_v, [j16], jnp.minimum(cur, cm),
                               mask=lane0)
            return best, bidx

        best, bidx = lax.fori_loop(
            jlo, jhi, inner,
            (inf16, jnp.zeros((16,), jnp.int32)))
        xg0 = plsc.load_gather(x0_v, [bidx])
        xg1 = plsc.load_gather(x1_v, [bidx])
        xg2 = plsc.load_gather(x2_v, [bidx])
        for yc, xg in ((y0, xg0), (y1, xg1), (y2, xg2)):
            err = yc - xg
            a = jnp.abs(err)
            hub16 = hub16 + jnp.where(a < 1.0, 0.5 * err * err, a - 0.5)
        return hub16

    hub16 = lax.fori_loop(0, _NVY, yblk, zero16)

    def dbody(k, carry):
        sp, spt, st = carry
        z = seg_v[pl.ds(k * 16, 16)]
        p = 1.0 / (1.0 + jnp.exp(-z))
        t = aty_v[pl.ds(k * 16, 16)]
        return sp + p, spt + p * t, st + t
    sp, spt, st = lax.fori_loop(0, _NVSEG, dbody, (zero16, zero16, zero16))

    part_v[pl.ds(0, 16)] = hub16
    part_v[pl.ds(16, 16)] = sp
    part_v[pl.ds(32, 16)] = spt
    part_v[pl.ds(48, 16)] = st
    pltpu.sync_copy(colmin_v, colmin_sh.at[sid])
    pltpu.sync_copy(part_v, part_sh.at[sid])
    plsc.subcore_barrier()

    for r in range(16):
        pltpu.sync_copy(colmin_sh.at[r, pl.ds(sid * 64, 64)],
                        mrg_v.at[pl.ds(r * 64, 64)])

    def mbody(r, carry):
        c0, c1, c2, c3 = carry
        c0 = jnp.minimum(c0, mrg_v[pl.ds(r * 64, 16)])
        c1 = jnp.minimum(c1, mrg_v[pl.ds(r * 64 + 16, 16)])
        c2 = jnp.minimum(c2, mrg_v[pl.ds(r * 64 + 32, 16)])
        c3 = jnp.minimum(c3, mrg_v[pl.ds(r * 64 + 48, 16)])
        return c0, c1, c2, c3
    c0, c1, c2, c3 = lax.fori_loop(0, 16, mbody, (inf16, inf16, inf16, inf16))
    cm_v[pl.ds(0, 16)] = c0
    cm_v[pl.ds(16, 16)] = c1
    cm_v[pl.ds(32, 16)] = c2
    cm_v[pl.ds(48, 16)] = c3
    pltpu.sync_copy(cm_v, colmin_out.at[pl.ds(cid * _N_X + sid * 64, 64)])

    @pl.when(sid == 0)
    def _merge_parts():
        for r in range(16):
            pltpu.sync_copy(part_sh.at[r], pall_v.at[pl.ds(r * 64, 64)])

        def pbody(r, carry):
            h, a, b, c = carry
            return (h + pall_v[pl.ds(r * 64, 16)],
                    a + pall_v[pl.ds(r * 64 + 16, 16)],
                    b + pall_v[pl.ds(r * 64 + 32, 16)],
                    c + pall_v[pl.ds(r * 64 + 48, 16)])
        h, a, b, c = lax.fori_loop(0, 16, pbody,
                                   (zero16, zero16, zero16, zero16))
        pout_v[pl.ds(0, 16)] = h
        pout_v[pl.ds(16, 16)] = a
        pout_v[pl.ds(32, 16)] = b
        pout_v[pl.ds(48, 16)] = c
        pltpu.sync_copy(pout_v, part_out.at[pl.ds(cid * 64, 64)])


def _fin_body(cm2, part2, conf, xt, y0c, out):
    cm = jnp.minimum(cm2[0, :], cm2[1, :])
    d2y0 = jnp.zeros((_N_X,), jnp.float32)
    for c in range(3):
        dd = xt[c, :] - y0c[c, 0]
        d2y0 = d2y0 + dd * dd
    cm = jnp.where(jnp.isinf(cm), d2y0, cm)
    dist = jnp.sqrt(cm)
    dc = conf[...] - dist
    conf_loss = jnp.sum(dc * dc) / _N_X
    p2 = part2[...]
    hub = jnp.sum(p2[0, :] + p2[4, :])
    sp = jnp.sum(p2[1, :] + p2[5, :])
    spt = jnp.sum(p2[2, :] + p2[6, :])
    st = jnp.sum(p2[3, :] + p2[7, :])
    eps = 1e-6
    dice = 1.0 - (2.0 * spt + eps) / (sp + st + eps)
    out[...] = (hub / (_N_Y * 3.0) + dice + conf_loss).reshape(1, 1)


@jax.jit
def kernel(pred_seg, atom_y, pred_pos_global_node, bindingsite_center,
           preds_confidence, x_batch, y_batch):
    xt = pred_pos_global_node.T
    x0 = xt[0]
    x1 = xt[1]
    x2 = xt[2]
    y3 = bindingsite_center.T.reshape(3, _NW, _YPW).transpose(1, 0, 2)
    yt_flat = y3.reshape(_NW, 3 * _YPW)
    yb2 = y_batch.reshape(_NW, _YPW)
    y0c = bindingsite_center.T[:, 0:1]
    seg2 = jnp.pad(pred_seg[:, 0], (0, _SEG_PAD - _N_ATOM),
                   constant_values=-88.0).reshape(_NW, _SEGW)
    aty2 = jnp.pad(atom_y, (0, _SEG_PAD - _N_ATOM)).reshape(_NW, _SEGW)
    conf = preds_confidence[:, 0]

    f32 = jnp.float32
    i32 = jnp.int32
    sc = pl.kernel(
        _sc_body,
        out_type=(jax.ShapeDtypeStruct((2 * _N_X,), f32),
                  jax.ShapeDtypeStruct((2 * 64,), f32)),
        mesh=plsc.VectorSubcoreMesh(core_axis_name="c", subcore_axis_name="s"),
        compiler_params=pltpu.CompilerParams(needs_layout_passes=False),
        scratch_types=[
            pltpu.VMEM((_N_X,), f32),
            pltpu.VMEM((_N_X,), f32),
            pltpu.VMEM((_N_X,), f32),
            pltpu.VMEM((_N_X,), i32),
            pltpu.VMEM((_B,), i32),
            pltpu.VMEM((_B,), i32),
            pltpu.VMEM((3 * _YPW,), f32),
            pltpu.VMEM((_YPW,), i32),
            pltpu.VMEM((_N_X,), f32),
            pltpu.VMEM((_SEGW,), f32),
            pltpu.VMEM((_SEGW,), f32),
            pltpu.VMEM((64,), f32),
            pltpu.VMEM((1024,), f32),
            pltpu.VMEM((64,), f32),
            pltpu.VMEM((1024,), f32),
            pltpu.VMEM((64,), f32),
            pltpu.VMEM_SHARED((16, _N_X), f32),
            pltpu.VMEM_SHARED((16, 64), f32),
        ],
    )
    colmin2, part2 = sc(x0, x1, x2, x_batch, yt_flat, yb2, seg2, aty2)

    out = pl.pallas_call(
        _fin_body,
        out_shape=jax.ShapeDtypeStruct((1, 1), f32),
    )(colmin2.reshape(2, _N_X), part2.reshape(8, 16), conf, xt, y0c)
    return out[0, 0]
